# Initial kernel scaffold; baseline (speedup 1.0000x reference)
import jax
import jax.numpy as jnp
from jax.experimental import pallas as pl


def kernel(node_2D_repr, positions, pos_noise, t_graph, params, batch, edge_index, anneal_power):
    raise NotImplementedError("write your pallas kernel here")


# speedup vs baseline: 1.4706x; 1.4706x over previous
import functools
import numpy as np

import jax
import jax.numpy as jnp
from jax import lax
from jax.experimental import pallas as pl
from jax.experimental.pallas import tpu as pltpu
from jax.experimental.pallas import tpu_sc as plsc

EPS = 1e-6
SIGMA_MIN = 0.1
SIGMA_MAX = 10.0
N = 10000
E = 160000
NG = 128
D = 256

F32 = jnp.float32


_NB = 1000


def _k1_body(x_ref, pos_ref, noise_ref, batch_ref, tg_ref,
             w1t_ref, w1b_ref, pa_ref, pb_ref,
             u_ref, v_ref, rt_ref, ct_ref):
    x = x_ref[...]
    u_ref[...] = jnp.dot(x, w1t_ref[...], preferred_element_type=F32)
    v_ref[...] = jnp.dot(x, w1b_ref[...], preferred_element_type=F32)
    qa = jnp.dot(x, pa_ref[...], preferred_element_type=F32)
    qb = jnp.dot(x, pb_ref[...], preferred_element_type=F32)
    t = tg_ref[...] * (1.0 - EPS) + EPS
    std_g = SIGMA_MIN * jnp.exp(t * np.log(SIGMA_MAX / SIGMA_MIN))
    lanes = lax.broadcasted_iota(jnp.int32, (1, NG), 1)
    onehot = (batch_ref[...] == lanes).astype(F32)
    std_n = jnp.sum(onehot * std_g, axis=1, keepdims=True)
    posp = pos_ref[...] + std_n * noise_ref[...]
    pad = jnp.zeros((_NB, 125), dtype=F32)
    rt_ref[...] = jnp.concatenate([qa, posp, pad], axis=1)
    ct_ref[...] = jnp.concatenate([qb, posp, pad], axis=1)


def _node_precompute(x, positions, pos_noise, batchi, tg, w1t, w1b, pa, pb):
    grid = (N // _NB,)
    fullspec = lambda shape: pl.BlockSpec(shape, lambda i: (0, 0))
    rowspec = lambda wdt: pl.BlockSpec((_NB, wdt), lambda i: (i, 0))
    return pl.pallas_call(
        _k1_body,
        grid=grid,
        in_specs=[rowspec(D), rowspec(3), rowspec(3), rowspec(1), fullspec((1, NG)),
                  fullspec((D, D)), fullspec((D, D)), fullspec((D, 128)), fullspec((D, 128))],
        out_specs=[rowspec(D), rowspec(D), rowspec(D), rowspec(D)],
        out_shape=[jax.ShapeDtypeStruct((N, D), F32), jax.ShapeDtypeStruct((N, D), F32),
                   jax.ShapeDtypeStruct((N, D), F32), jax.ShapeDtypeStruct((N, D), F32)],
    )(x, positions, pos_noise, batchi, tg, w1t, w1b, pa, pb)


_NW = 32
_EPW = E // _NW
_CB = 40


def _k2_body(u_hbm, v_hbm, rt_hbm, ct_hbm, row_hbm, col_hbm,
             h_out, qpr_out, pc_out,
             idxr, idxc, bu, bv, brt, bct, s0, s1, s2, s3):
    wid = lax.axis_index("s") * 2 + lax.axis_index("c")
    base0 = wid * _EPW

    def chunk(ch, carry):
        base = base0 + ch * _CB
        pltpu.sync_copy(row_hbm.at[pl.ds(base, _CB)], idxr)
        pltpu.sync_copy(col_hbm.at[pl.ds(base, _CB)], idxc)
        c0 = pltpu.async_copy(u_hbm.at[idxr], bu, s0)
        c1 = pltpu.async_copy(v_hbm.at[idxc], bv, s1)
        c2 = pltpu.async_copy(rt_hbm.at[idxr], brt, s2)
        c3 = pltpu.async_copy(ct_hbm.at[idxc], bct, s3)
        c0.wait(); c1.wait(); c2.wait(); c3.wait()

        def erow(e, c):
            for k in range(D // 16):
                sl = pl.ds(k * 16, 16)
                bu[e, sl] = bu[e, sl] + bv[e, sl]
            for k in range(128 // 16):
                sl = pl.ds(k * 16, 16)
                brt[e, sl] = brt[e, sl] + bct[e, sl]
            return c
        lax.fori_loop(0, _CB, erow, 0)
        pltpu.sync_copy(bu, h_out.at[pl.ds(base, _CB)])
        pltpu.sync_copy(brt, qpr_out.at[pl.ds(base, _CB)])
        pltpu.sync_copy(bct.at[:, pl.ds(128, 128)], pc_out.at[pl.ds(base, _CB)])
        return carry

    lax.fori_loop(0, _EPW // _CB, chunk, 0)


def _edge_gather(u, v, rt, ct, row, col):
    mesh = plsc.VectorSubcoreMesh(core_axis_name="c", subcore_axis_name="s")
    f = pl.kernel(
        _k2_body,
        out_type=[jax.ShapeDtypeStruct((E, D), F32),
                  jax.ShapeDtypeStruct((E, D), F32),
                  jax.ShapeDtypeStruct((E, 128), F32)],
        mesh=mesh,
        scratch_types=[
            pltpu.VMEM((_CB,), jnp.int32), pltpu.VMEM((_CB,), jnp.int32),
            pltpu.VMEM((_CB, D), F32), pltpu.VMEM((_CB, D), F32),
            pltpu.VMEM((_CB, D), F32), pltpu.VMEM((_CB, D), F32),
            pltpu.SemaphoreType.DMA, pltpu.SemaphoreType.DMA,
            pltpu.SemaphoreType.DMA, pltpu.SemaphoreType.DMA,
        ],
    )
    return f(u, v, rt, ct, row, col)


_SB = 8000


def _k3_body(h_ref, s1_ref, s2_ref):
    i = pl.program_id(0)

    @pl.when(i == 0)
    def _():
        s1_ref[...] = jnp.zeros_like(s1_ref)
        s2_ref[...] = jnp.zeros_like(s2_ref)

    h = h_ref[...]
    s1 = jnp.sum(h, axis=0, keepdims=True)
    s2 = jnp.sum(h * h, axis=0, keepdims=True)
    s1_ref[...] += jnp.broadcast_to(s1, (8, D))
    s2_ref[...] += jnp.broadcast_to(s2, (8, D))


def _bn_stats(h):
    return pl.pallas_call(
        _k3_body,
        grid=(E // _SB,),
        in_specs=[pl.BlockSpec((_SB, D), lambda i: (i, 0))],
        out_specs=[pl.BlockSpec((8, D), lambda i: (0, 0)),
                   pl.BlockSpec((8, D), lambda i: (0, 0))],
        out_shape=[jax.ShapeDtypeStruct((8, D), F32), jax.ShapeDtypeStruct((8, D), F32)],
    )(h)


_EB = 800


def _silu(x):
    return x / (1.0 + jnp.exp(-x))


def _k4_body(h_ref, qpr_ref, pc_ref, s1_ref, s2_ref,
             w2_ref, b2_ref, inws_ref, inwc_ref, inb_ref, distw_ref, coffw_ref,
             c1_ref, c2_ref, a0_ref, a1_ref, bproj_ref, pw2_ref, pb2_ref,
             w1c_ref, bq_ref, snw2_ref, snb2_ref,
             ev_ref):
    mu = s1_ref[0:1, :] * (1.0 / E)
    ex2 = s2_ref[0:1, :] * (1.0 / E)
    var = ex2 - mu * mu
    rstd = lax.rsqrt(var + 1e-5)

    hn = jnp.maximum((h_ref[...] - mu) * rstd, 0.0)
    attr2 = jnp.dot(hn, w2_ref[...], preferred_element_type=F32) + b2_ref[...]

    qpr = qpr_ref[...]
    pcb = pc_ref[...]
    q = qpr[:, 0:128]
    prx = qpr[:, 128:129]; pry = qpr[:, 129:130]; prz = qpr[:, 130:131]
    pcx = pcb[:, 0:1]; pcy = pcb[:, 1:2]; pcz = pcb[:, 2:3]

    dxx = prx - pcx; dxy = pry - pcy; dxz = prz - pcz
    radial = dxx * dxx + dxy * dxy + dxz * dxz
    dist = jnp.sqrt(radial)
    nrm = dist + EPS
    inv_nrm = 1.0 / nrm
    cdx = dxx * inv_nrm; cdy = dxy * inv_nrm; cdz = dxz * inv_nrm

    crx = pry * pcz - prz * pcy
    cry = prz * pcx - prx * pcz
    crz = prx * pcy - pry * pcx
    cn = jnp.sqrt(crx * crx + cry * cry + crz * crz) + EPS
    inv_cn = 1.0 / cn
    ccx = crx * inv_cn; ccy = cry * inv_cn; ccz = crz * inv_cn

    vx = cdy * ccz - cdz * ccy
    vy = cdz * ccx - cdx * ccz
    vz = cdx * ccy - cdy * ccx

    ci0 = cdx * prx + cdy * pry + cdz * prz
    cj0 = cdx * pcx + cdy * pcy + cdz * pcz
    ci1 = ccx * prx + ccy * pry + ccz * prz
    cj1 = ccx * pcx + ccy * pcy + ccz * pcz
    ci2 = vx * prx + vy * pry + vz * prz
    cj2 = vx * pcx + vy * pcy + vz * pcz

    ni = jnp.sqrt(ci0 * ci0 + ci1 * ci1 + ci2 * ci2)
    nj = jnp.sqrt(cj0 * cj0 + cj1 * cj1 + cj2 * cj2)
    pcos = ((ci0 * cj0 + jnp.abs(ci1) * jnp.abs(cj1) + ci2 * cj2)
            / (ni + EPS) / (nj + EPS))
    psin = jnp.sqrt(jnp.maximum(1.0 - pcos * pcos, 0.0))

    tw = 2.0 * np.pi
    ang = dist * (distw_ref[...] * tw)
    attr3 = (jnp.dot(jnp.sin(ang), inws_ref[...], preferred_element_type=F32)
             + jnp.dot(jnp.cos(ang), inwc_ref[...], preferred_element_type=F32)
             + inb_ref[...])

    cw = coffw_ref[...] * tw
    acc = psin * a0_ref[...] + pcos * a1_ref[...] + bproj_ref[...]
    c1 = c1_ref[...]
    c2 = c2_ref[...]
    for (cmat, s0, s2v) in ((c1, ci0, ci2), (c2, cj0, cj2)):
        a_lo = s0 * cw
        a_hi = s2v * cw
        acc += jnp.dot(jnp.sin(a_lo), cmat[0:D, :], preferred_element_type=F32)
        acc += jnp.dot(jnp.cos(a_lo), cmat[D:2 * D, :], preferred_element_type=F32)
        acc += jnp.dot(jnp.sin(a_hi), cmat[2 * D:3 * D, :], preferred_element_type=F32)
        acc += jnp.dot(jnp.cos(a_hi), cmat[3 * D:4 * D, :], preferred_element_type=F32)
    ph = _silu(acc)
    frame = jnp.dot(ph, pw2_ref[...], preferred_element_type=F32) + pb2_ref[...]
    edge_attr = attr3 * attr2 + frame

    s = q + jnp.dot(edge_attr, w1c_ref[...], preferred_element_type=F32) + bq_ref[...]
    mh = _silu(s)
    c3 = jnp.dot(mh, snw2_ref[...], preferred_element_type=F32) + snb2_ref[...]
    c30 = c3[:, 0:1]; c31 = c3[:, 1:2]; c32 = c3[:, 2:3]
    evx = c30 * cdx + c31 * ccx + c32 * vx
    evy = c30 * cdy + c31 * ccy + c32 * vy
    evz = c30 * cdz + c31 * ccz + c32 * vz
    ev_ref[...] = jnp.concatenate(
        [evx, evy, evz, jnp.zeros((_EB, 125), dtype=F32)], axis=1)


def _edge_compute(h, qpr, pc, s1, s2, weights):
    grid = (E // _EB,)
    full = lambda shape: pl.BlockSpec(shape, lambda i: tuple(0 for _ in shape))
    rows = lambda wdt: pl.BlockSpec((_EB, wdt), lambda i: (i, 0))
    in_specs = [rows(D), rows(D), rows(128), full((8, D)), full((8, D))]
    in_specs += [full(w.shape) for w in weights]
    return pl.pallas_call(
        _k4_body,
        grid=grid,
        in_specs=in_specs,
        out_specs=[pl.BlockSpec((_EB, 128), lambda i: (i, 0))],
        out_shape=[jax.ShapeDtypeStruct((E, 128), F32)],
    )(h, qpr, pc, s1, s2, *weights)[0]


_SCB = 80
_NSPLIT = 5056
_NSH = 5120
_ZPT = _NSH // 16


def _k5_body(ev_hbm, row_hbm, g_out, idxv, idx2, bev, zb, shared):
    cid = lax.axis_index("c")
    sid = lax.axis_index("s")
    base_node = cid * _NSPLIT

    def zrow(i, c):
        for k in range(8):
            zb[i, pl.ds(k * 16, 16)] = jnp.zeros((16,), dtype=F32)
        return c
    lax.fori_loop(0, _ZPT, zrow, 0)
    pltpu.sync_copy(zb, shared.at[pl.ds(sid * _ZPT, _ZPT)])
    plsc.subcore_barrier()

    base0 = sid * (E // 16)

    def chunk(ch, carry):
        base = base0 + ch * _SCB
        pltpu.sync_copy(row_hbm.at[pl.ds(base, _SCB)], idxv)
        pltpu.sync_copy(ev_hbm.at[pl.ds(base, _SCB)], bev)
        for k in range(_SCB // 16):
            sl = pl.ds(k * 16, 16)
            r = idxv[sl] - base_node
            ok = (r >= 0) & (r < _NSPLIT)
            idx2[sl] = jnp.where(ok, r, _NSPLIT)
        pltpu.sync_copy(bev, shared.at[idx2], add=True)
        return carry

    lax.fori_loop(0, (E // 16) // _SCB, chunk, 0)
    plsc.subcore_barrier()

    @pl.when(sid < 15)
    def _():
        pltpu.sync_copy(shared.at[pl.ds(sid * 320, 320)], zb)
        pltpu.sync_copy(zb, g_out.at[cid, pl.ds(sid * 320, 320)])

    @pl.when(sid == 15)
    def _():
        pltpu.sync_copy(shared.at[pl.ds(4800, 256)], zb.at[pl.ds(0, 256)])
        pltpu.sync_copy(zb.at[pl.ds(0, 256)], g_out.at[cid, pl.ds(4800, 256)])


def _segment_scatter(ev, row):
    mesh = plsc.VectorSubcoreMesh(core_axis_name="c", subcore_axis_name="s")
    f = pl.kernel(
        _k5_body,
        out_type=jax.ShapeDtypeStruct((2, _NSPLIT, 128), F32),
        mesh=mesh,
        scratch_types=[
            pltpu.VMEM((_SCB,), jnp.int32),
            pltpu.VMEM((_SCB,), jnp.int32),
            pltpu.VMEM((_SCB, 128), F32),
            pltpu.VMEM((_ZPT, 128), F32),
            pltpu.VMEM_SHARED((_NSH, 128), F32),
        ],
    )
    return f(ev, row)


def kernel(node_2D_repr, positions, pos_noise, t_graph, params, batch, edge_index, anneal_power):
    p = params
    w1t, w1b = p['e2d_W1'][:D], p['e2d_W1'][D:]
    pa = p['node_W'] @ p['sn_W1'][0:D]
    pb = p['node_W'] @ p['sn_W1'][D:2 * D]
    w1c = p['sn_W1'][2 * D:3 * D]
    bias_q = (p['sn_b1'] + p['node_b'] @ p['sn_W1'][0:D]
              + p['node_b'] @ p['sn_W1'][D:2 * D])[None, :]
    c1 = p['coffmlp_W'] @ p['proj_W1'][2:2 + D]
    c2 = p['coffmlp_W'] @ p['proj_W1'][2 + D:2 + 2 * D]
    b_proj = (p['proj_b1'] + p['coffmlp_b'] @ (p['proj_W1'][2:2 + D]
                                               + p['proj_W1'][2 + D:2 + 2 * D]))[None, :]
    a0 = p['proj_W1'][0:1]
    a1 = p['proj_W1'][1:2]
    inws, inwc = p['in_W'][:D], p['in_W'][D:]
    snw2 = jnp.zeros((128, 128), F32).at[:, 0:3].set(p['sn_W2'])
    snb2 = jnp.zeros((1, 128), F32).at[:, 0:3].set(p['sn_b2'][None, :])

    batchi = batch[:, None]
    tg = t_graph[None, :]
    row = edge_index[0]
    col = edge_index[1]

    u, v, rt, ct = _node_precompute(
        node_2D_repr, positions, pos_noise, batchi, tg, w1t, w1b, pa, pb)
    h, qpr, pc = _edge_gather(u, v, rt, ct, row, col)
    s1, s2 = _bn_stats(h)
    weights = [p['e2d_W2'], p['e2d_b2'][None, :], inws, inwc, p['in_b'][None, :],
               p['dist_W'][None, :], p['coff_W'][None, :],
               c1, c2, a0, a1, b_proj, p['proj_W2'], p['proj_b2'][None, :],
               w1c, bias_q, snw2, snb2]
    ev = _edge_compute(h, qpr, pc, s1, s2, weights)
    g = _segment_scatter(ev, row).reshape(2 * _NSPLIT, 128)
    return g[0:N, 0:3]


# speedup vs baseline: 2.3276x; 1.5828x over previous
import functools
import numpy as np

import jax
import jax.numpy as jnp
from jax import lax
from jax.experimental import pallas as pl
from jax.experimental.pallas import tpu as pltpu
from jax.experimental.pallas import tpu_sc as plsc

EPS = 1e-6
SIGMA_MIN = 0.1
SIGMA_MAX = 10.0
N = 10000
E = 160000
NG = 128
D = 256

F32 = jnp.float32


_NB = 1000


def _k1_body(x_ref, pos_ref, noise_ref, batch_ref, tg_ref,
             w1t_ref, w1b_ref, pa_ref, pb_ref,
             u_ref, v_ref, rt_ref, ct_ref):
    x = x_ref[...]
    u_ref[...] = jnp.dot(x, w1t_ref[...], preferred_element_type=F32)
    v_ref[...] = jnp.dot(x, w1b_ref[...], preferred_element_type=F32)
    qa = jnp.dot(x, pa_ref[...], preferred_element_type=F32)
    qb = jnp.dot(x, pb_ref[...], preferred_element_type=F32)
    t = tg_ref[...] * (1.0 - EPS) + EPS
    std_g = SIGMA_MIN * jnp.exp(t * np.log(SIGMA_MAX / SIGMA_MIN))
    lanes = lax.broadcasted_iota(jnp.int32, (1, NG), 1)
    onehot = (batch_ref[...] == lanes).astype(F32)
    std_n = jnp.sum(onehot * std_g, axis=1, keepdims=True)
    posp = pos_ref[...] + std_n * noise_ref[...]
    pad = jnp.zeros((_NB, 125), dtype=F32)
    rt_ref[...] = jnp.concatenate([qa, posp, pad], axis=1)
    ct_ref[...] = jnp.concatenate([qb, posp, pad], axis=1)


def _node_precompute(x, positions, pos_noise, batchi, tg, w1t, w1b, pa, pb):
    grid = (N // _NB,)
    fullspec = lambda shape: pl.BlockSpec(shape, lambda i: (0, 0))
    rowspec = lambda wdt: pl.BlockSpec((_NB, wdt), lambda i: (i, 0))
    return pl.pallas_call(
        _k1_body,
        grid=grid,
        in_specs=[rowspec(D), rowspec(3), rowspec(3), rowspec(1), fullspec((1, NG)),
                  fullspec((D, D)), fullspec((D, D)), fullspec((D, 128)), fullspec((D, 128))],
        out_specs=[rowspec(D), rowspec(D), rowspec(D), rowspec(D)],
        out_shape=[jax.ShapeDtypeStruct((N, D), F32), jax.ShapeDtypeStruct((N, D), F32),
                   jax.ShapeDtypeStruct((N, D), F32), jax.ShapeDtypeStruct((N, D), F32)],
    )(x, positions, pos_noise, batchi, tg, w1t, w1b, pa, pb)


_NW = 32
_EPW = E // _NW
_CB = 40


def _k2_body(u_hbm, v_hbm, rt_hbm, ct_hbm, row_hbm, col_hbm,
             h_out, qpr_out, pc_out,
             idxr, idxc, bu, bv, brt, bct, s0, s1, s2, s3):
    wid = lax.axis_index("s") * 2 + lax.axis_index("c")
    base0 = wid * _EPW

    def chunk(ch, carry):
        base = base0 + ch * _CB
        pltpu.sync_copy(row_hbm.at[pl.ds(base, _CB)], idxr)
        pltpu.sync_copy(col_hbm.at[pl.ds(base, _CB)], idxc)
        c0 = pltpu.async_copy(u_hbm.at[idxr], bu, s0)
        c1 = pltpu.async_copy(v_hbm.at[idxc], bv, s1)
        c2 = pltpu.async_copy(rt_hbm.at[idxr], brt, s2)
        c3 = pltpu.async_copy(ct_hbm.at[idxc], bct, s3)
        c0.wait(); c1.wait(); c2.wait(); c3.wait()

        def erow(e, c):
            for k in range(D // 16):
                sl = pl.ds(k * 16, 16)
                bu[e, sl] = bu[e, sl] + bv[e, sl]
            for k in range(128 // 16):
                sl = pl.ds(k * 16, 16)
                brt[e, sl] = brt[e, sl] + bct[e, sl]
            return c
        lax.fori_loop(0, _CB, erow, 0)
        pltpu.sync_copy(bu, h_out.at[pl.ds(base, _CB)])
        pltpu.sync_copy(brt, qpr_out.at[pl.ds(base, _CB)])
        pltpu.sync_copy(bct.at[:, pl.ds(128, 128)], pc_out.at[pl.ds(base, _CB)])
        return carry

    lax.fori_loop(0, _EPW // _CB, chunk, 0)


def _edge_gather(u, v, rt, ct, row, col):
    mesh = plsc.VectorSubcoreMesh(core_axis_name="c", subcore_axis_name="s")
    f = pl.kernel(
        _k2_body,
        out_type=[jax.ShapeDtypeStruct((E, D), F32),
                  jax.ShapeDtypeStruct((E, D), F32),
                  jax.ShapeDtypeStruct((E, 128), F32)],
        mesh=mesh,
        scratch_types=[
            pltpu.VMEM((_CB,), jnp.int32), pltpu.VMEM((_CB,), jnp.int32),
            pltpu.VMEM((_CB, D), F32), pltpu.VMEM((_CB, D), F32),
            pltpu.VMEM((_CB, D), F32), pltpu.VMEM((_CB, D), F32),
            pltpu.SemaphoreType.DMA, pltpu.SemaphoreType.DMA,
            pltpu.SemaphoreType.DMA, pltpu.SemaphoreType.DMA,
        ],
    )
    return f(u, v, rt, ct, row, col)


_SB = 8000


def _k3_body(h_ref, s1_ref, s2_ref):
    i = pl.program_id(0)

    @pl.when(i == 0)
    def _():
        s1_ref[...] = jnp.zeros_like(s1_ref)
        s2_ref[...] = jnp.zeros_like(s2_ref)

    h = h_ref[...]
    s1 = jnp.sum(h, axis=0, keepdims=True)
    s2 = jnp.sum(h * h, axis=0, keepdims=True)
    s1_ref[...] += jnp.broadcast_to(s1, (8, D))
    s2_ref[...] += jnp.broadcast_to(s2, (8, D))


def _bn_stats(h):
    return pl.pallas_call(
        _k3_body,
        grid=(E // _SB,),
        in_specs=[pl.BlockSpec((_SB, D), lambda i: (i, 0))],
        out_specs=[pl.BlockSpec((8, D), lambda i: (0, 0)),
                   pl.BlockSpec((8, D), lambda i: (0, 0))],
        out_shape=[jax.ShapeDtypeStruct((8, D), F32), jax.ShapeDtypeStruct((8, D), F32)],
    )(h)


_EB = 800


def _silu(x):
    return x / (1.0 + jnp.exp(-x))


_SIN_C = (0.9999999827737748, -0.16666651514235015, 0.008332963909001756,
          -0.00019804748134769412, 2.5980951125369577e-06)
_COS_C = (0.9999999998456133, -0.4999999951142117, 0.04166664187638778,
          -0.001388843233082876, 2.47637666162959e-05, -2.611494973412389e-07)
_INV_PI = float(1.0 / np.pi)
_PI_HI = 3.140625
_PI_LO = float(np.pi - 3.140625)


def _fast_sincos(x):
    k = jnp.round(x * _INV_PI)
    r = (x - k * _PI_HI) - k * _PI_LO
    p = k * 0.5
    sgn = 1.0 - 4.0 * (p - jnp.floor(p))
    r2 = r * r
    s = _SIN_C[4]
    for c in _SIN_C[3::-1]:
        s = s * r2 + c
    s = s * r
    co = _COS_C[5]
    for c in _COS_C[4::-1]:
        co = co * r2 + c
    return s * sgn, co * sgn


def _k4_body(h_ref, qpr_ref, pc_ref, s1_ref, s2_ref,
             w2_ref, b2_ref, inws_ref, inwc_ref, inb_ref, distw_ref, coffw_ref,
             c1_ref, c2_ref, a0_ref, a1_ref, bproj_ref, pw2_ref, pb2_ref,
             w1c_ref, bq_ref, snw2_ref, snb2_ref,
             ev_ref):
    mu = s1_ref[0:1, :] * (1.0 / E)
    ex2 = s2_ref[0:1, :] * (1.0 / E)
    var = ex2 - mu * mu
    rstd = lax.rsqrt(var + 1e-5)

    hn = jnp.maximum((h_ref[...] - mu) * rstd, 0.0)
    attr2 = jnp.dot(hn, w2_ref[...], preferred_element_type=F32) + b2_ref[...]

    qpr = qpr_ref[...]
    pcb = pc_ref[...]
    q = qpr[:, 0:128]
    prx = qpr[:, 128:129]; pry = qpr[:, 129:130]; prz = qpr[:, 130:131]
    pcx = pcb[:, 0:1]; pcy = pcb[:, 1:2]; pcz = pcb[:, 2:3]

    dxx = prx - pcx; dxy = pry - pcy; dxz = prz - pcz
    radial = dxx * dxx + dxy * dxy + dxz * dxz
    dist = jnp.sqrt(radial)
    nrm = dist + EPS
    inv_nrm = 1.0 / nrm
    cdx = dxx * inv_nrm; cdy = dxy * inv_nrm; cdz = dxz * inv_nrm

    crx = pry * pcz - prz * pcy
    cry = prz * pcx - prx * pcz
    crz = prx * pcy - pry * pcx
    cn = jnp.sqrt(crx * crx + cry * cry + crz * crz) + EPS
    inv_cn = 1.0 / cn
    ccx = crx * inv_cn; ccy = cry * inv_cn; ccz = crz * inv_cn

    vx = cdy * ccz - cdz * ccy
    vy = cdz * ccx - cdx * ccz
    vz = cdx * ccy - cdy * ccx

    ci0 = cdx * prx + cdy * pry + cdz * prz
    cj0 = cdx * pcx + cdy * pcy + cdz * pcz
    ci1 = ccx * prx + ccy * pry + ccz * prz
    cj1 = ccx * pcx + ccy * pcy + ccz * pcz
    ci2 = vx * prx + vy * pry + vz * prz
    cj2 = vx * pcx + vy * pcy + vz * pcz

    ni = jnp.sqrt(ci0 * ci0 + ci1 * ci1 + ci2 * ci2)
    nj = jnp.sqrt(cj0 * cj0 + cj1 * cj1 + cj2 * cj2)
    pcos = ((ci0 * cj0 + jnp.abs(ci1) * jnp.abs(cj1) + ci2 * cj2)
            / (ni + EPS) / (nj + EPS))
    psin = jnp.sqrt(jnp.maximum(1.0 - pcos * pcos, 0.0))

    tw = 2.0 * np.pi
    ang = dist * (distw_ref[...] * tw)
    sin_d, cos_d = _fast_sincos(ang)
    attr3 = (jnp.dot(sin_d, inws_ref[...], preferred_element_type=F32)
             + jnp.dot(cos_d, inwc_ref[...], preferred_element_type=F32)
             + inb_ref[...])

    cw = coffw_ref[...] * tw
    acc = psin * a0_ref[...] + pcos * a1_ref[...] + bproj_ref[...]
    c1 = c1_ref[...]
    c2 = c2_ref[...]
    for (cmat, s0, s2v) in ((c1, ci0, ci2), (c2, cj0, cj2)):
        sin_lo, cos_lo = _fast_sincos(s0 * cw)
        sin_hi, cos_hi = _fast_sincos(s2v * cw)
        acc += jnp.dot(sin_lo, cmat[0:D, :], preferred_element_type=F32)
        acc += jnp.dot(cos_lo, cmat[D:2 * D, :], preferred_element_type=F32)
        acc += jnp.dot(sin_hi, cmat[2 * D:3 * D, :], preferred_element_type=F32)
        acc += jnp.dot(cos_hi, cmat[3 * D:4 * D, :], preferred_element_type=F32)
    ph = _silu(acc)
    frame = jnp.dot(ph, pw2_ref[...], preferred_element_type=F32) + pb2_ref[...]
    edge_attr = attr3 * attr2 + frame

    s = q + jnp.dot(edge_attr, w1c_ref[...], preferred_element_type=F32) + bq_ref[...]
    mh = _silu(s)
    c3 = jnp.dot(mh, snw2_ref[...], preferred_element_type=F32) + snb2_ref[...]
    c30 = c3[:, 0:1]; c31 = c3[:, 1:2]; c32 = c3[:, 2:3]
    evx = c30 * cdx + c31 * ccx + c32 * vx
    evy = c30 * cdy + c31 * ccy + c32 * vy
    evz = c30 * cdz + c31 * ccz + c32 * vz
    ev_ref[...] = jnp.concatenate(
        [evx, evy, evz, jnp.zeros((_EB, 125), dtype=F32)], axis=1)


def _edge_compute(h, qpr, pc, s1, s2, weights):
    grid = (E // _EB,)
    full = lambda shape: pl.BlockSpec(shape, lambda i: tuple(0 for _ in shape))
    rows = lambda wdt: pl.BlockSpec((_EB, wdt), lambda i: (i, 0))
    in_specs = [rows(D), rows(D), rows(128), full((8, D)), full((8, D))]
    in_specs += [full(w.shape) for w in weights]
    return pl.pallas_call(
        _k4_body,
        grid=grid,
        in_specs=in_specs,
        out_specs=[pl.BlockSpec((_EB, 128), lambda i: (i, 0))],
        out_shape=[jax.ShapeDtypeStruct((E, 128), F32)],
    )(h, qpr, pc, s1, s2, *weights)[0]


_SCB = 80
_NSPLIT = 5056
_NSH = 5120
_ZPT = _NSH // 16


def _k5_body(ev_hbm, row_hbm, g_out, idxv, idx2, bev, zb, shared):
    cid = lax.axis_index("c")
    sid = lax.axis_index("s")
    base_node = cid * _NSPLIT

    def zrow(i, c):
        for k in range(8):
            zb[i, pl.ds(k * 16, 16)] = jnp.zeros((16,), dtype=F32)
        return c
    lax.fori_loop(0, _ZPT, zrow, 0)
    pltpu.sync_copy(zb, shared.at[pl.ds(sid * _ZPT, _ZPT)])
    plsc.subcore_barrier()

    base0 = sid * (E // 16)

    def chunk(ch, carry):
        base = base0 + ch * _SCB
        pltpu.sync_copy(row_hbm.at[pl.ds(base, _SCB)], idxv)
        pltpu.sync_copy(ev_hbm.at[pl.ds(base, _SCB)], bev)
        for k in range(_SCB // 16):
            sl = pl.ds(k * 16, 16)
            r = idxv[sl] - base_node
            ok = (r >= 0) & (r < _NSPLIT)
            idx2[sl] = jnp.where(ok, r, _NSPLIT)
        pltpu.sync_copy(bev, shared.at[idx2], add=True)
        return carry

    lax.fori_loop(0, (E // 16) // _SCB, chunk, 0)
    plsc.subcore_barrier()

    @pl.when(sid < 15)
    def _():
        pltpu.sync_copy(shared.at[pl.ds(sid * 320, 320)], zb)
        pltpu.sync_copy(zb, g_out.at[cid, pl.ds(sid * 320, 320)])

    @pl.when(sid == 15)
    def _():
        pltpu.sync_copy(shared.at[pl.ds(4800, 256)], zb.at[pl.ds(0, 256)])
        pltpu.sync_copy(zb.at[pl.ds(0, 256)], g_out.at[cid, pl.ds(4800, 256)])


def _segment_scatter(ev, row):
    mesh = plsc.VectorSubcoreMesh(core_axis_name="c", subcore_axis_name="s")
    f = pl.kernel(
        _k5_body,
        out_type=jax.ShapeDtypeStruct((2, _NSPLIT, 128), F32),
        mesh=mesh,
        scratch_types=[
            pltpu.VMEM((_SCB,), jnp.int32),
            pltpu.VMEM((_SCB,), jnp.int32),
            pltpu.VMEM((_SCB, 128), F32),
            pltpu.VMEM((_ZPT, 128), F32),
            pltpu.VMEM_SHARED((_NSH, 128), F32),
        ],
    )
    return f(ev, row)


def kernel(node_2D_repr, positions, pos_noise, t_graph, params, batch, edge_index, anneal_power):
    p = params
    w1t, w1b = p['e2d_W1'][:D], p['e2d_W1'][D:]
    pa = p['node_W'] @ p['sn_W1'][0:D]
    pb = p['node_W'] @ p['sn_W1'][D:2 * D]
    w1c = p['sn_W1'][2 * D:3 * D]
    bias_q = (p['sn_b1'] + p['node_b'] @ p['sn_W1'][0:D]
              + p['node_b'] @ p['sn_W1'][D:2 * D])[None, :]
    c1 = p['coffmlp_W'] @ p['proj_W1'][2:2 + D]
    c2 = p['coffmlp_W'] @ p['proj_W1'][2 + D:2 + 2 * D]
    b_proj = (p['proj_b1'] + p['coffmlp_b'] @ (p['proj_W1'][2:2 + D]
                                               + p['proj_W1'][2 + D:2 + 2 * D]))[None, :]
    a0 = p['proj_W1'][0:1]
    a1 = p['proj_W1'][1:2]
    inws, inwc = p['in_W'][:D], p['in_W'][D:]
    snw2 = jnp.zeros((128, 128), F32).at[:, 0:3].set(p['sn_W2'])
    snb2 = jnp.zeros((1, 128), F32).at[:, 0:3].set(p['sn_b2'][None, :])

    batchi = batch[:, None]
    tg = t_graph[None, :]
    row = edge_index[0]
    col = edge_index[1]

    u, v, rt, ct = _node_precompute(
        node_2D_repr, positions, pos_noise, batchi, tg, w1t, w1b, pa, pb)
    h, qpr, pc = _edge_gather(u, v, rt, ct, row, col)
    s1, s2 = _bn_stats(h)
    weights = [p['e2d_W2'], p['e2d_b2'][None, :], inws, inwc, p['in_b'][None, :],
               p['dist_W'][None, :], p['coff_W'][None, :],
               c1, c2, a0, a1, b_proj, p['proj_W2'], p['proj_b2'][None, :],
               w1c, bias_q, snw2, snb2]
    ev = _edge_compute(h, qpr, pc, s1, s2, weights)
    g = _segment_scatter(ev, row).reshape(2 * _NSPLIT, 128)
    return g[0:N, 0:3]
